# baseline (speedup 1.0000x reference)
import functools

import jax
import jax.numpy as jnp
from jax import lax
from jax.experimental import pallas as pl
from jax.experimental.pallas import tpu as pltpu
from jax.experimental.pallas import tpu_sc as plsc

B = 4
S = 2048
D = 768
LANES = 16
D_VECS = D // LANES

NUM_CORES = 2
NUM_SUBCORES = 16
NW = NUM_CORES * NUM_SUBCORES
S_PER_W = S // NW
SCHUNK = 8
ROWS = SCHUNK * B
NSTEP = S_PER_W // SCHUNK
NBUF = 3
RPW = B * S_PER_W


def _make_kernel():
    mesh = plsc.VectorSubcoreMesh(core_axis_name="c", subcore_axis_name="s")

    @functools.partial(
        pl.kernel,
        mesh=mesh,
        out_type=jax.ShapeDtypeStruct((B * S, D), jnp.float32),
        scratch_types=[
            pltpu.VMEM((RPW,), jnp.int32),
            pltpu.VMEM((NSTEP, ROWS), jnp.int32),
            pltpu.VMEM((S_PER_W, D), jnp.float32),
            pltpu.VMEM((ROWS, D), jnp.float32),
            pltpu.VMEM((ROWS, D), jnp.float32),
            pltpu.VMEM((ROWS, D), jnp.float32),
            pltpu.SemaphoreType.DMA,
            pltpu.SemaphoreType.DMA,
            pltpu.SemaphoreType.DMA,
            pltpu.SemaphoreType.DMA,
            pltpu.SemaphoreType.DMA,
            pltpu.SemaphoreType.DMA,
            pltpu.SemaphoreType.DMA,
            pltpu.SemaphoreType.DMA,
            pltpu.SemaphoreType.DMA,
        ],
    )
    def emb_kernel(xr_hbm, oidx_hbm, tok_hbm, pos_hbm, out_hbm,
                   idx_v, oidx_v, pos_v, t0, t1, t2,
                   gs0, gs1, gs2, os0, os1, os2, psem, isem, osem2):
        wid = lax.axis_index("s") * NUM_CORES + lax.axis_index("c")
        s_base = wid * S_PER_W

        toks = (t0, t1, t2)
        gss = (gs0, gs1, gs2)
        oss = (os0, os1, os2)

        icp = pltpu.async_copy(xr_hbm.at[pl.ds(wid * RPW, RPW)], idx_v, isem)
        ocp = pltpu.async_copy(
            oidx_hbm.at[pl.ds(wid * NSTEP, NSTEP)], oidx_v, osem2)
        pcp = pltpu.async_copy(pos_hbm.at[pl.ds(s_base, S_PER_W)],
                               pos_v, psem)
        icp.wait()
        ocp.wait()

        def start_step(t):
            p = t % NBUF
            pltpu.async_copy(
                tok_hbm.at[idx_v.at[pl.ds(t * ROWS, ROWS)]],
                toks[p], gss[p])

        def process_step(t):
            p = t % NBUF
            pltpu.make_async_copy(
                tok_hbm.at[idx_v.at[pl.ds(t * ROWS, ROWS)]],
                toks[p], gss[p]).wait()
            pbase = t * SCHUNK

            def add_pos(r, c2):
                for c in range(D_VECS):
                    sl = pl.ds(c * LANES, LANES)
                    pvec = pos_v[pbase + r, sl]
                    for bb in range(B):
                        plsc.addupdate(toks[p].at[r * B + bb, sl], pvec)
                return c2

            lax.fori_loop(0, SCHUNK, add_pos, 0)
            pltpu.async_copy(toks[p], out_hbm.at[oidx_v.at[t]], oss[p])

        def wait_out(p, t):
            pltpu.make_async_copy(toks[p], out_hbm.at[oidx_v.at[t]],
                                  oss[p]).wait()

        start_step(0)
        start_step(1)
        pcp.wait()
        for t in range(NSTEP):
            process_step(t)
            if t + 2 < NSTEP:
                if t >= 1:
                    wait_out((t + 2) % NBUF, t - 1)
                start_step(t + 2)
        for t in range(NSTEP - NBUF, NSTEP):
            wait_out(t % NBUF, t)

    return emb_kernel


_emb_kernel = _make_kernel()


def kernel(x, token_emb, pos_emb):
    xr = (x.astype(jnp.int32)
           .reshape(B, NW, S_PER_W)
           .transpose(1, 2, 0)
           .reshape(-1))
    s_idx = jnp.arange(S, dtype=jnp.int32)
    b_idx = jnp.arange(B, dtype=jnp.int32)
    oidx = (b_idx[None, :] * S + s_idx[:, None])
    oidx = oidx.reshape(NW * NSTEP, ROWS)
    out = _emb_kernel(xr, oidx, token_emb, pos_emb)
    return out.reshape(B, S, D)


# speedup vs baseline: 1.1394x; 1.1394x over previous
import functools

import jax
import jax.numpy as jnp
from jax import lax
from jax.experimental import pallas as pl
from jax.experimental.pallas import tpu as pltpu
from jax.experimental.pallas import tpu_sc as plsc

B = 4
S = 2048
D = 768
LANES = 16
D_VECS = D // LANES

NUM_CORES = 2
NUM_SUBCORES = 16
NW = NUM_CORES * NUM_SUBCORES
S_PER_W = S // NW
SCHUNK = 8
ROWS = SCHUNK * B
NSTEP = S_PER_W // SCHUNK
NBUF = 3
RPW = B * S_PER_W


def _make_kernel():
    mesh = plsc.VectorSubcoreMesh(core_axis_name="c", subcore_axis_name="s")

    @functools.partial(
        pl.kernel,
        mesh=mesh,
        out_type=jax.ShapeDtypeStruct((B * S, D), jnp.float32),
        scratch_types=[
            pltpu.VMEM((RPW,), jnp.int32),
            pltpu.VMEM((NSTEP, ROWS), jnp.int32),
            pltpu.VMEM((S_PER_W, D), jnp.float32),
            pltpu.VMEM((ROWS, D), jnp.float32),
            pltpu.VMEM((ROWS, D), jnp.float32),
            pltpu.VMEM((ROWS, D), jnp.float32),
            pltpu.SemaphoreType.DMA,
            pltpu.SemaphoreType.DMA,
            pltpu.SemaphoreType.DMA,
            pltpu.SemaphoreType.DMA,
            pltpu.SemaphoreType.DMA,
            pltpu.SemaphoreType.DMA,
            pltpu.SemaphoreType.DMA,
            pltpu.SemaphoreType.DMA,
            pltpu.SemaphoreType.DMA,
        ],
    )
    def emb_kernel(xr_hbm, oidx_hbm, tok_hbm, pos_hbm, out_hbm,
                   idx_v, oidx_v, pos_v, t0, t1, t2,
                   gs0, gs1, gs2, os0, os1, os2, psem, isem, osem2):
        wid = lax.axis_index("s") * NUM_CORES + lax.axis_index("c")
        s_base = wid * S_PER_W

        toks = (t0, t1, t2)
        gss = (gs0, gs1, gs2)
        oss = (os0, os1, os2)

        icp = pltpu.async_copy(xr_hbm.at[pl.ds(wid * RPW, RPW)], idx_v, isem)
        ocp = pltpu.async_copy(
            oidx_hbm.at[pl.ds(wid * NSTEP, NSTEP)], oidx_v, osem2)
        pcp = pltpu.async_copy(pos_hbm.at[pl.ds(s_base, S_PER_W)],
                               pos_v, psem)
        icp.wait()
        ocp.wait()

        def start_step(t):
            p = t % NBUF
            pltpu.async_copy(
                tok_hbm.at[idx_v.at[pl.ds(t * ROWS, ROWS)]],
                toks[p], gss[p])

        def process_step(t):
            p = t % NBUF
            pltpu.make_async_copy(
                tok_hbm.at[idx_v.at[pl.ds(t * ROWS, ROWS)]],
                toks[p], gss[p]).wait()
            pbase = t * SCHUNK

            @plsc.parallel_loop(0, SCHUNK, unroll=2)
            def add_pos(r):
                for c in range(D_VECS):
                    sl = pl.ds(c * LANES, LANES)
                    pvec = pos_v[pbase + r, sl]
                    for bb in range(B):
                        plsc.addupdate(toks[p].at[r * B + bb, sl], pvec)
            pltpu.async_copy(toks[p], out_hbm.at[oidx_v.at[t]], oss[p])

        def wait_out(p, t):
            pltpu.make_async_copy(toks[p], out_hbm.at[oidx_v.at[t]],
                                  oss[p]).wait()

        start_step(0)
        start_step(1)
        pcp.wait()
        for t in range(NSTEP):
            process_step(t)
            if t + 2 < NSTEP:
                if t >= 1:
                    wait_out((t + 2) % NBUF, t - 1)
                start_step(t + 2)
        for t in range(NSTEP - NBUF, NSTEP):
            wait_out(t % NBUF, t)

    return emb_kernel


_emb_kernel = _make_kernel()


def kernel(x, token_emb, pos_emb):
    xr = (x.astype(jnp.int32)
           .reshape(B, NW, S_PER_W)
           .transpose(1, 2, 0)
           .reshape(-1))
    s_idx = jnp.arange(S, dtype=jnp.int32)
    b_idx = jnp.arange(B, dtype=jnp.int32)
    oidx = (b_idx[None, :] * S + s_idx[:, None])
    oidx = oidx.reshape(NW * NSTEP, ROWS)
    out = _emb_kernel(xr, oidx, token_emb, pos_emb)
    return out.reshape(B, S, D)


# speedup vs baseline: 1.1477x; 1.0073x over previous
import functools

import numpy as np

import jax
import jax.numpy as jnp
from jax import lax
from jax.experimental import pallas as pl
from jax.experimental.pallas import tpu as pltpu
from jax.experimental.pallas import tpu_sc as plsc

B = 4
S = 2048
D = 768
LANES = 16
D_VECS = D // LANES

NUM_CORES = 2
NUM_SUBCORES = 16
NW = NUM_CORES * NUM_SUBCORES
S_PER_W = S // NW
SCHUNK = 8
ROWS = SCHUNK * B
NSTEP = S_PER_W // SCHUNK
NBUF = 3
RPW = B * S_PER_W


def _make_kernel():
    mesh = plsc.VectorSubcoreMesh(core_axis_name="c", subcore_axis_name="s")

    @functools.partial(
        pl.kernel,
        mesh=mesh,
        out_type=jax.ShapeDtypeStruct((B * S, D), jnp.float32),
        scratch_types=[
            pltpu.VMEM((RPW,), jnp.int32),
            pltpu.VMEM((NSTEP, ROWS), jnp.int32),
            pltpu.VMEM((S_PER_W, D), jnp.float32),
            pltpu.VMEM((ROWS, D), jnp.float32),
            pltpu.VMEM((ROWS, D), jnp.float32),
            pltpu.VMEM((ROWS, D), jnp.float32),
            pltpu.SemaphoreType.DMA,
            pltpu.SemaphoreType.DMA,
            pltpu.SemaphoreType.DMA,
            pltpu.SemaphoreType.DMA,
            pltpu.SemaphoreType.DMA,
            pltpu.SemaphoreType.DMA,
            pltpu.SemaphoreType.DMA,
            pltpu.SemaphoreType.DMA,
            pltpu.SemaphoreType.DMA,
        ],
    )
    def emb_kernel(xr_hbm, oidx_hbm, tok_hbm, pos_hbm, out_hbm,
                   idx_v, oidx_v, pos_v, t0, t1, t2,
                   gs0, gs1, gs2, os0, os1, os2, psem, isem, osem2):
        wid = lax.axis_index("s") * NUM_CORES + lax.axis_index("c")
        s_base = wid * S_PER_W

        toks = (t0, t1, t2)
        gss = (gs0, gs1, gs2)
        oss = (os0, os1, os2)

        icp = pltpu.async_copy(xr_hbm.at[pl.ds(wid * RPW, RPW)], idx_v, isem)
        ocp = pltpu.async_copy(
            oidx_hbm.at[pl.ds(wid * NSTEP, NSTEP)], oidx_v, osem2)
        pcp = pltpu.async_copy(pos_hbm.at[pl.ds(s_base, S_PER_W)],
                               pos_v, psem)
        icp.wait()
        ocp.wait()

        def start_step(t):
            p = t % NBUF
            pltpu.async_copy(
                tok_hbm.at[idx_v.at[pl.ds(t * ROWS, ROWS)]],
                toks[p], gss[p])

        def process_step(t):
            p = t % NBUF
            pltpu.make_async_copy(
                tok_hbm.at[idx_v.at[pl.ds(t * ROWS, ROWS)]],
                toks[p], gss[p]).wait()
            pbase = t * SCHUNK

            @plsc.parallel_loop(0, SCHUNK, unroll=2)
            def add_pos(r):
                for c in range(D_VECS):
                    sl = pl.ds(c * LANES, LANES)
                    pvec = pos_v[pbase + r, sl]
                    for bb in range(B):
                        plsc.addupdate(toks[p].at[r * B + bb, sl], pvec)
            pltpu.async_copy(toks[p], out_hbm.at[oidx_v.at[t]], oss[p])

        def wait_out(p, t):
            pltpu.make_async_copy(toks[p], out_hbm.at[oidx_v.at[t]],
                                  oss[p]).wait()

        start_step(0)
        start_step(1)
        pcp.wait()
        for t in range(NSTEP):
            process_step(t)
            if t + 2 < NSTEP:
                if t >= 1:
                    wait_out((t + 2) % NBUF, t - 1)
                start_step(t + 2)
        for t in range(NSTEP - NBUF, NSTEP):
            wait_out(t % NBUF, t)

    return emb_kernel


_emb_kernel = _make_kernel()


_OIDX = jnp.asarray(
    (np.arange(B, dtype=np.int32)[None, :] * S
     + np.arange(S, dtype=np.int32)[:, None]).reshape(NW * NSTEP, ROWS))


def kernel(x, token_emb, pos_emb):
    xr = (x.astype(jnp.int32)
           .reshape(B, NW, S_PER_W)
           .transpose(1, 2, 0)
           .reshape(-1))
    oidx = _OIDX
    out = _emb_kernel(xr, oidx, token_emb, pos_emb)
    return out.reshape(B, S, D)
